# Initial kernel scaffold; baseline (speedup 1.0000x reference)
import functools

import jax
import jax.numpy as jnp
from jax import lax
from jax.experimental import pallas as pl
from jax.experimental.pallas import tpu as pltpu
from jax.experimental.pallas import tpu_sc as plsc

N = 10000
E = 320000
D = 128
DEPTH = 4

NC = 2
NS = 16
NW = NC * NS
EPW = E // NW
CH = 80
NCHUNK = EPW // CH
RPT = N // NS

_mesh = plsc.VectorSubcoreMesh(core_axis_name="c", subcore_axis_name="s")


@functools.partial(
    pl.kernel,
    out_type=jax.ShapeDtypeStruct((NC, N, 16), jnp.float32),
    mesh=_mesh,
    scratch_types=[
        pltpu.VMEM((NCHUNK, CH), jnp.int32),
        pltpu.VMEM((CH, 16), jnp.float32),
        pltpu.VMEM_SHARED((N, 16), jnp.float32),
    ],
)
def _sc_degree(dst_hbm, zeros_hbm, ones_hbm, deg_out, didx_v, ones_v, dacc_sh):
    c = lax.axis_index("c")
    s = lax.axis_index("s")
    wid = s * NC + c
    pltpu.sync_copy(dst_hbm.at[wid], didx_v)
    pltpu.sync_copy(ones_hbm, ones_v)
    pltpu.sync_copy(zeros_hbm, dacc_sh.at[pl.ds(s * RPT, RPT)])
    plsc.subcore_barrier()

    def chunk(j, _):
        pltpu.sync_copy(ones_v, dacc_sh.at[didx_v.at[j]], add=True)
        return 0

    lax.fori_loop(0, NCHUNK, chunk, 0)
    plsc.subcore_barrier()
    pltpu.sync_copy(dacc_sh.at[pl.ds(s * RPT, RPT)],
                    deg_out.at[c, pl.ds(s * RPT, RPT)])


@functools.partial(
    pl.kernel,
    out_type=jax.ShapeDtypeStruct((NC, N, D), jnp.float32),
    mesh=_mesh,
    scratch_types=[
        pltpu.VMEM((NCHUNK, CH), jnp.int32),
        pltpu.VMEM((NCHUNK, CH), jnp.int32),
        pltpu.VMEM((CH, D), jnp.float32),
        pltpu.VMEM_SHARED((N, D), jnp.float32),
        pltpu.SemaphoreType.DMA,
    ],
)
def _sc_aggregate(hn_hbm, src_hbm, dst_hbm, zeros_hbm, out_hbm,
                  sidx_v, didx_v, gbuf, acc_sh, sem):
    c = lax.axis_index("c")
    s = lax.axis_index("s")
    wid = s * NC + c
    pltpu.sync_copy(src_hbm.at[wid], sidx_v)
    pltpu.sync_copy(dst_hbm.at[wid], didx_v)
    pltpu.sync_copy(zeros_hbm, acc_sh.at[pl.ds(s * RPT, RPT)])
    plsc.subcore_barrier()

    def chunk(j, _):
        pltpu.async_copy(hn_hbm.at[sidx_v.at[j]], gbuf, sem).wait()
        pltpu.sync_copy(gbuf, acc_sh.at[didx_v.at[j]], add=True)
        return 0

    lax.fori_loop(0, NCHUNK, chunk, 0)
    plsc.subcore_barrier()
    pltpu.sync_copy(acc_sh.at[pl.ds(s * RPT, RPT)],
                    out_hbm.at[c, pl.ds(s * RPT, RPT)])


_RB = 1000
_GRID = N // _RB


def _tc_proj_in_body(h_ref, w_ref, b_ref, deg_ref, hn_ref, nb_ref):
    deg = deg_ref[0, :, 0] + deg_ref[1, :, 0]
    norm = lax.rsqrt(jnp.maximum(deg, 1.0))[:, None]
    h0 = jnp.dot(h_ref[...], w_ref[...],
                 preferred_element_type=jnp.float32) + b_ref[...]
    normb = jnp.broadcast_to(norm, (_RB, D))
    hn_ref[...] = h0 * normb
    nb_ref[...] = normb


def _tc_layer_body(parts_ref, nb_ref, w_ref, b_ref, hn_ref):
    agg = (parts_ref[0] + parts_ref[1]) * nb_ref[...]
    t = jnp.dot(agg, w_ref[...], preferred_element_type=jnp.float32) + b_ref[...]
    h = t * jax.nn.sigmoid(t)
    hn_ref[...] = h * nb_ref[...]


def _tc_last_body(parts_ref, nb_ref, w_ref, b_ref, wo_ref, bo_ref, out_ref):
    agg = (parts_ref[0] + parts_ref[1]) * nb_ref[...]
    t = jnp.dot(agg, w_ref[...], preferred_element_type=jnp.float32) + b_ref[...]
    h = t * jax.nn.sigmoid(t)
    out_ref[...] = jnp.dot(h, wo_ref[...],
                           preferred_element_type=jnp.float32) + bo_ref[...]


_spec_rows = pl.BlockSpec((_RB, D), lambda i: (i, 0))
_spec_parts = pl.BlockSpec((NC, _RB, D), lambda i: (0, i, 0))
_spec_w = pl.BlockSpec((D, D), lambda i: (0, 0))
_spec_b = pl.BlockSpec((1, D), lambda i: (0, 0))
_spec_deg = pl.BlockSpec((NC, _RB, 16), lambda i: (0, i, 0))

_proj_in_call = pl.pallas_call(
    _tc_proj_in_body,
    grid=(_GRID,),
    in_specs=[_spec_rows, _spec_w, _spec_b, _spec_deg],
    out_specs=[_spec_rows, _spec_rows],
    out_shape=[jax.ShapeDtypeStruct((N, D), jnp.float32),
               jax.ShapeDtypeStruct((N, D), jnp.float32)],
)

_layer_call = pl.pallas_call(
    _tc_layer_body,
    grid=(_GRID,),
    in_specs=[_spec_parts, _spec_rows, _spec_w, _spec_b],
    out_specs=_spec_rows,
    out_shape=jax.ShapeDtypeStruct((N, D), jnp.float32),
)

_last_call = pl.pallas_call(
    _tc_last_body,
    grid=(_GRID,),
    in_specs=[_spec_parts, _spec_rows, _spec_w, _spec_b, _spec_w, _spec_b],
    out_specs=_spec_rows,
    out_shape=jax.ShapeDtypeStruct((N, D), jnp.float32),
)


@jax.jit
def kernel(h, edge_index, W_in, b_in, W_layers, b_layers, W_out, b_out):
    src = edge_index[0].reshape(NW, NCHUNK, CH)
    dst = edge_index[1].reshape(NW, NCHUNK, CH)
    zeros_d = jnp.zeros((RPT, D), jnp.float32)
    zeros16 = jnp.zeros((RPT, 16), jnp.float32)
    ones16 = jnp.ones((CH, 16), jnp.float32)

    deg_parts = _sc_degree(dst, zeros16, ones16)
    hn, normb = _proj_in_call(h, W_in, b_in.reshape(1, D), deg_parts)
    for i in range(DEPTH - 1):
        parts = _sc_aggregate(hn, src, dst, zeros_d)
        hn = _layer_call(parts, normb, W_layers[i], b_layers[i].reshape(1, D))
    parts = _sc_aggregate(hn, src, dst, zeros_d)
    out = _last_call(parts, normb, W_layers[DEPTH - 1],
                     b_layers[DEPTH - 1].reshape(1, D),
                     W_out, b_out.reshape(1, D))
    return out


# speedup vs baseline: 6.1584x; 6.1584x over previous
import functools

import jax
import jax.numpy as jnp
from jax import lax
from jax.experimental import pallas as pl
from jax.experimental.pallas import tpu as pltpu
from jax.experimental.pallas import tpu_sc as plsc

N = 10000
NP = 10240
E = 320000
D = 128
DEPTH = 4

NC = 2
NS = 16
NW = NC * NS
EPW = E // NW
CH = 80
NCHUNK = EPW // CH
RPT = NP // NS

_mesh = plsc.VectorSubcoreMesh(core_axis_name="c", subcore_axis_name="s")


def _sc_degree_body(dst_hbm, zeros_hbm, ones_hbm, deg_out, didx_v, ones_v, dacc_sh):
    c = lax.axis_index("c")
    s = lax.axis_index("s")
    wid = s * NC + c
    pltpu.sync_copy(dst_hbm.at[wid], didx_v)
    pltpu.sync_copy(ones_hbm, ones_v)
    pltpu.sync_copy(zeros_hbm, dacc_sh.at[pl.ds(s * RPT, RPT)])
    plsc.subcore_barrier()

    def chunk(j, _):
        pltpu.sync_copy(ones_v, dacc_sh.at[didx_v.at[j]], add=True)
        return 0

    lax.fori_loop(0, NCHUNK, chunk, 0)
    plsc.subcore_barrier()
    pltpu.sync_copy(dacc_sh.at[pl.ds(s * RPT, RPT)],
                    deg_out.at[c, pl.ds(s * RPT, RPT)])


def _sc_aggregate_body(hn_hbm, src_hbm, dst_hbm, zeros_hbm, out_hbm,
                  sidx_v, didx_v, gbuf, acc_sh, sem):
    c = lax.axis_index("c")
    s = lax.axis_index("s")
    wid = s * NC + c
    pltpu.sync_copy(src_hbm.at[wid], sidx_v)
    pltpu.sync_copy(dst_hbm.at[wid], didx_v)
    pltpu.sync_copy(zeros_hbm, acc_sh.at[pl.ds(s * RPT, RPT)])
    plsc.subcore_barrier()

    def chunk(j, _):
        pltpu.async_copy(hn_hbm.at[sidx_v.at[j]], gbuf, sem).wait()
        pltpu.sync_copy(gbuf, acc_sh.at[didx_v.at[j]], add=True)
        return 0

    lax.fori_loop(0, NCHUNK, chunk, 0)
    plsc.subcore_barrier()
    pltpu.sync_copy(acc_sh.at[pl.ds(s * RPT, RPT)],
                    out_hbm.at[c, pl.ds(s * RPT, RPT)])


_sc_degree = functools.partial(
    pl.kernel,
    out_type=jax.ShapeDtypeStruct((NC, NP, D), jnp.float32),
    mesh=_mesh,
    scratch_types=[
        pltpu.VMEM((NCHUNK, CH), jnp.int32),
        pltpu.VMEM((CH, D), jnp.float32),
        pltpu.VMEM_SHARED((NP, D), jnp.float32),
    ],
)(_sc_degree_body)

_sc_aggregate = functools.partial(
    pl.kernel,
    out_type=jax.ShapeDtypeStruct((NC, NP, D), jnp.float32),
    mesh=_mesh,
    scratch_types=[
        pltpu.VMEM((NCHUNK, CH), jnp.int32),
        pltpu.VMEM((NCHUNK, CH), jnp.int32),
        pltpu.VMEM((CH, D), jnp.float32),
        pltpu.VMEM_SHARED((NP, D), jnp.float32),
        pltpu.SemaphoreType.DMA,
    ],
)(_sc_aggregate_body)


_RB = 1024
_GRID = NP // _RB


def _tc_proj_in_body(h_ref, w_ref, b_ref, deg_ref, hn_ref, nb_ref):
    deg = deg_ref[0, :, 0] + deg_ref[1, :, 0]
    norm = lax.rsqrt(jnp.maximum(deg, 1.0))[:, None]
    h0 = jnp.dot(h_ref[...], w_ref[...],
                 preferred_element_type=jnp.float32) + b_ref[...]
    normb = jnp.broadcast_to(norm, (_RB, D))
    hn_ref[...] = h0 * normb
    nb_ref[...] = normb


def _tc_layer_body(parts_ref, nb_ref, w_ref, b_ref, hn_ref):
    agg = (parts_ref[0] + parts_ref[1]) * nb_ref[...]
    t = jnp.dot(agg, w_ref[...], preferred_element_type=jnp.float32) + b_ref[...]
    h = t * jax.nn.sigmoid(t)
    hn_ref[...] = h * nb_ref[...]


def _tc_last_body(parts_ref, nb_ref, w_ref, b_ref, wo_ref, bo_ref, out_ref):
    agg = (parts_ref[0] + parts_ref[1]) * nb_ref[...]
    t = jnp.dot(agg, w_ref[...], preferred_element_type=jnp.float32) + b_ref[...]
    h = t * jax.nn.sigmoid(t)
    out_ref[...] = jnp.dot(h, wo_ref[...],
                           preferred_element_type=jnp.float32) + bo_ref[...]


_spec_rows = pl.BlockSpec((_RB, D), lambda i: (i, 0))
_spec_parts = pl.BlockSpec((NC, _RB, D), lambda i: (0, i, 0))
_spec_w = pl.BlockSpec((D, D), lambda i: (0, 0))
_spec_b = pl.BlockSpec((1, D), lambda i: (0, 0))
_spec_deg = pl.BlockSpec((NC, _RB, D), lambda i: (0, i, 0))

_proj_in_call = pl.pallas_call(
    _tc_proj_in_body,
    grid=(_GRID,),
    in_specs=[_spec_rows, _spec_w, _spec_b, _spec_deg],
    out_specs=[_spec_rows, _spec_rows],
    out_shape=[jax.ShapeDtypeStruct((NP, D), jnp.float32),
               jax.ShapeDtypeStruct((NP, D), jnp.float32)],
)

_layer_call = pl.pallas_call(
    _tc_layer_body,
    grid=(_GRID,),
    in_specs=[_spec_parts, _spec_rows, _spec_w, _spec_b],
    out_specs=_spec_rows,
    out_shape=jax.ShapeDtypeStruct((NP, D), jnp.float32),
)

_last_call = pl.pallas_call(
    _tc_last_body,
    grid=(_GRID,),
    in_specs=[_spec_parts, _spec_rows, _spec_w, _spec_b, _spec_w, _spec_b],
    out_specs=_spec_rows,
    out_shape=jax.ShapeDtypeStruct((NP, D), jnp.float32),
)


@jax.jit
def kernel(h, edge_index, W_in, b_in, W_layers, b_layers, W_out, b_out):
    src = edge_index[0].reshape(NW, NCHUNK, CH)
    dst = edge_index[1].reshape(NW, NCHUNK, CH)
    zeros_d = jnp.zeros((RPT, D), jnp.float32)
    ones_d = jnp.ones((CH, D), jnp.float32)
    hp = jnp.pad(h, ((0, NP - N), (0, 0)))

    deg_parts = _sc_degree(dst, zeros_d, ones_d)
    hn, normb = _proj_in_call(hp, W_in, b_in.reshape(1, D), deg_parts)
    for i in range(DEPTH - 1):
        parts = _sc_aggregate(hn, src, dst, zeros_d)
        hn = _layer_call(parts, normb, W_layers[i], b_layers[i].reshape(1, D))
    parts = _sc_aggregate(hn, src, dst, zeros_d)
    out = _last_call(parts, normb, W_layers[DEPTH - 1],
                     b_layers[DEPTH - 1].reshape(1, D),
                     W_out, b_out.reshape(1, D))
    return out[:N]


# speedup vs baseline: 7.8336x; 1.2720x over previous
import functools

import jax
import jax.numpy as jnp
from jax import lax
from jax.experimental import pallas as pl
from jax.experimental.pallas import tpu as pltpu
from jax.experimental.pallas import tpu_sc as plsc

N = 10000
NP = 10240
E = 320000
D = 128
DEPTH = 4

NC = 2
NS = 16
NW = NC * NS
EPW = E // NW
CH = 80
NCHUNK = EPW // CH
RPT = NP // NS

_mesh = plsc.VectorSubcoreMesh(core_axis_name="c", subcore_axis_name="s")


def _sc_degree_body(dst_hbm, zeros_hbm, ones_hbm, deg_out, didx_v, ones_v, dacc_sh):
    c = lax.axis_index("c")
    s = lax.axis_index("s")
    wid = s * NC + c
    pltpu.sync_copy(dst_hbm.at[wid], didx_v)
    pltpu.sync_copy(ones_hbm, ones_v)
    pltpu.sync_copy(zeros_hbm, dacc_sh.at[pl.ds(s * RPT, RPT)])
    plsc.subcore_barrier()

    def chunk(j, _):
        pltpu.sync_copy(ones_v, dacc_sh.at[didx_v.at[j]], add=True)
        return 0

    lax.fori_loop(0, NCHUNK, chunk, 0)
    plsc.subcore_barrier()
    pltpu.sync_copy(dacc_sh.at[pl.ds(s * RPT, RPT)],
                    deg_out.at[c, pl.ds(s * RPT, RPT)])


def _sc_aggregate_body(hn_hbm, src_hbm, dst_hbm, zeros_hbm, out_hbm,
                  sidx_v, didx_v, buf0, buf1, acc_sh,
                  gsem0, gsem1, ssem0, ssem1):
    c = lax.axis_index("c")
    s = lax.axis_index("s")
    wid = s * NC + c
    pltpu.sync_copy(src_hbm.at[wid], sidx_v)

    def sidx(j):
        return sidx_v.at[pl.ds(j * CH, CH)]

    pltpu.async_copy(hn_hbm.at[sidx(0)], buf0, gsem0)
    pltpu.async_copy(hn_hbm.at[sidx(1)], buf1, gsem1)
    pltpu.sync_copy(dst_hbm.at[wid], didx_v)
    pltpu.sync_copy(zeros_hbm, acc_sh.at[pl.ds(s * RPT, RPT)])
    plsc.subcore_barrier()

    def pair(g, _):
        pltpu.make_async_copy(hn_hbm.at[sidx(g)], buf0, gsem0).wait()
        pltpu.async_copy(buf0, acc_sh.at[didx_v.at[g]], ssem0, add=True)
        pltpu.make_async_copy(hn_hbm.at[sidx(g + 1)], buf1, gsem1).wait()
        pltpu.async_copy(buf1, acc_sh.at[didx_v.at[g + 1]], ssem1, add=True)
        pltpu.make_async_copy(buf0, acc_sh.at[didx_v.at[g]], ssem0).wait()

        @pl.when(g + 2 < NCHUNK)
        def _():
            pltpu.async_copy(hn_hbm.at[sidx(g + 2)], buf0, gsem0)

        pltpu.make_async_copy(buf1, acc_sh.at[didx_v.at[g]], ssem1).wait()

        @pl.when(g + 3 < NCHUNK)
        def _():
            pltpu.async_copy(hn_hbm.at[sidx(g + 3)], buf1, gsem1)

        return 0

    lax.fori_loop(0, NCHUNK // 2, lambda i, x: pair(i * 2, x), 0)
    pltpu.make_async_copy(hn_hbm.at[sidx(NCHUNK - 1)], buf0, gsem0).wait()
    pltpu.async_copy(buf0, acc_sh.at[didx_v.at[NCHUNK - 1]], ssem0, add=True)
    pltpu.make_async_copy(buf0, acc_sh.at[didx_v.at[NCHUNK - 1]], ssem0).wait()
    plsc.subcore_barrier()
    pltpu.sync_copy(acc_sh.at[pl.ds(s * RPT, RPT)],
                    out_hbm.at[c, pl.ds(s * RPT, RPT)])


_sc_degree = functools.partial(
    pl.kernel,
    out_type=jax.ShapeDtypeStruct((NC, NP, D), jnp.float32),
    mesh=_mesh,
    scratch_types=[
        pltpu.VMEM((NCHUNK, CH), jnp.int32),
        pltpu.VMEM((CH, D), jnp.float32),
        pltpu.VMEM_SHARED((NP, D), jnp.float32),
    ],
)(_sc_degree_body)

_sc_aggregate = functools.partial(
    pl.kernel,
    out_type=jax.ShapeDtypeStruct((NC, NP, D), jnp.float32),
    mesh=_mesh,
    scratch_types=[
        pltpu.VMEM((EPW,), jnp.int32),
        pltpu.VMEM((NCHUNK, CH), jnp.int32),
        pltpu.VMEM((CH, D), jnp.float32),
        pltpu.VMEM((CH, D), jnp.float32),
        pltpu.VMEM_SHARED((NP, D), jnp.float32),
        pltpu.SemaphoreType.DMA,
        pltpu.SemaphoreType.DMA,
        pltpu.SemaphoreType.DMA,
        pltpu.SemaphoreType.DMA,
    ],
)(_sc_aggregate_body)


_RB = 1024
_GRID = NP // _RB


def _tc_proj_in_body(h_ref, w_ref, b_ref, deg_ref, hn_ref, nb_ref):
    deg = deg_ref[0, :, 0] + deg_ref[1, :, 0]
    norm = lax.rsqrt(jnp.maximum(deg, 1.0))[:, None]
    h0 = jnp.dot(h_ref[...], w_ref[...],
                 preferred_element_type=jnp.float32) + b_ref[...]
    normb = jnp.broadcast_to(norm, (_RB, D))
    hn_ref[...] = h0 * normb
    nb_ref[...] = normb


def _tc_layer_body(parts_ref, nb_ref, w_ref, b_ref, hn_ref):
    agg = (parts_ref[0] + parts_ref[1]) * nb_ref[...]
    t = jnp.dot(agg, w_ref[...], preferred_element_type=jnp.float32) + b_ref[...]
    h = t * jax.nn.sigmoid(t)
    hn_ref[...] = h * nb_ref[...]


def _tc_last_body(parts_ref, nb_ref, w_ref, b_ref, wo_ref, bo_ref, out_ref):
    agg = (parts_ref[0] + parts_ref[1]) * nb_ref[...]
    t = jnp.dot(agg, w_ref[...], preferred_element_type=jnp.float32) + b_ref[...]
    h = t * jax.nn.sigmoid(t)
    out_ref[...] = jnp.dot(h, wo_ref[...],
                           preferred_element_type=jnp.float32) + bo_ref[...]


_spec_rows = pl.BlockSpec((_RB, D), lambda i: (i, 0))
_spec_parts = pl.BlockSpec((NC, _RB, D), lambda i: (0, i, 0))
_spec_w = pl.BlockSpec((D, D), lambda i: (0, 0))
_spec_b = pl.BlockSpec((1, D), lambda i: (0, 0))
_spec_deg = pl.BlockSpec((NC, _RB, D), lambda i: (0, i, 0))

_proj_in_call = pl.pallas_call(
    _tc_proj_in_body,
    grid=(_GRID,),
    in_specs=[_spec_rows, _spec_w, _spec_b, _spec_deg],
    out_specs=[_spec_rows, _spec_rows],
    out_shape=[jax.ShapeDtypeStruct((NP, D), jnp.float32),
               jax.ShapeDtypeStruct((NP, D), jnp.float32)],
)

_layer_call = pl.pallas_call(
    _tc_layer_body,
    grid=(_GRID,),
    in_specs=[_spec_parts, _spec_rows, _spec_w, _spec_b],
    out_specs=_spec_rows,
    out_shape=jax.ShapeDtypeStruct((NP, D), jnp.float32),
)

_last_call = pl.pallas_call(
    _tc_last_body,
    grid=(_GRID,),
    in_specs=[_spec_parts, _spec_rows, _spec_w, _spec_b, _spec_w, _spec_b],
    out_specs=_spec_rows,
    out_shape=jax.ShapeDtypeStruct((NP, D), jnp.float32),
)


@jax.jit
def kernel(h, edge_index, W_in, b_in, W_layers, b_layers, W_out, b_out):
    src = edge_index[0].reshape(NW, EPW)
    dst = edge_index[1].reshape(NW, NCHUNK, CH)
    zeros_d = jnp.zeros((RPT, D), jnp.float32)
    ones_d = jnp.ones((CH, D), jnp.float32)
    hp = jnp.pad(h, ((0, NP - N), (0, 0)))

    deg_parts = _sc_degree(dst, zeros_d, ones_d)
    hn, normb = _proj_in_call(hp, W_in, b_in.reshape(1, D), deg_parts)
    for i in range(DEPTH - 1):
        parts = _sc_aggregate(hn, src, dst, zeros_d)
        hn = _layer_call(parts, normb, W_layers[i], b_layers[i].reshape(1, D))
    parts = _sc_aggregate(hn, src, dst, zeros_d)
    out = _last_call(parts, normb, W_layers[DEPTH - 1],
                     b_layers[DEPTH - 1].reshape(1, D),
                     W_out, b_out.reshape(1, D))
    return out[:N]


# speedup vs baseline: 8.5421x; 1.0904x over previous
import functools

import jax
import jax.numpy as jnp
from jax import lax
from jax.experimental import pallas as pl
from jax.experimental.pallas import tpu as pltpu
from jax.experimental.pallas import tpu_sc as plsc

N = 10000
NP = 10240
E = 320000
D = 128
DEPTH = 4

NC = 2
NS = 16
NW = NC * NS
EPW = E // NW
CH = 80
NCHUNK = EPW // CH
RPT = NP // NS

_mesh = plsc.VectorSubcoreMesh(core_axis_name="c", subcore_axis_name="s")


def _sc_degree_body(dst_hbm, deg_out, didx_v, cnt_v):
    c = lax.axis_index("c")
    s = lax.axis_index("s")
    wid = s * NC + c
    pltpu.sync_copy(dst_hbm.at[wid], didx_v)
    zeros16 = jnp.zeros((16,), jnp.float32)
    ones16 = jnp.ones((16,), jnp.float32)

    def zero(i, _):
        cnt_v[pl.ds(i * 16, 16)] = zeros16
        return 0

    lax.fori_loop(0, NP // 16, zero, 0)

    def hist(i, _):
        dvec = didx_v[pl.ds(i * 16, 16)]
        plsc.addupdate_scatter(cnt_v, [dvec], ones16)
        return 0

    lax.fori_loop(0, EPW // 16, hist, 0)
    pltpu.sync_copy(cnt_v, deg_out.at[wid])


def _sc_aggregate_body(hn_hbm, src_hbm, dst_hbm, zeros_hbm, out_hbm,
                  sidx_v, didx_v, buf0, buf1, acc_sh,
                  gsem0, gsem1, ssem0, ssem1):
    c = lax.axis_index("c")
    s = lax.axis_index("s")
    wid = s * NC + c
    pltpu.sync_copy(src_hbm.at[wid], sidx_v)

    def sidx(j):
        return sidx_v.at[pl.ds(j * CH, CH)]

    pltpu.async_copy(hn_hbm.at[sidx(0)], buf0, gsem0)
    pltpu.async_copy(hn_hbm.at[sidx(1)], buf1, gsem1)
    pltpu.sync_copy(dst_hbm.at[wid], didx_v)
    pltpu.sync_copy(zeros_hbm, acc_sh.at[pl.ds(s * RPT, RPT)])
    plsc.subcore_barrier()

    def pair(g, _):
        pltpu.make_async_copy(hn_hbm.at[sidx(g)], buf0, gsem0).wait()
        pltpu.async_copy(buf0, acc_sh.at[didx_v.at[g]], ssem0, add=True)
        pltpu.make_async_copy(hn_hbm.at[sidx(g + 1)], buf1, gsem1).wait()
        pltpu.async_copy(buf1, acc_sh.at[didx_v.at[g + 1]], ssem1, add=True)
        pltpu.make_async_copy(buf0, acc_sh.at[didx_v.at[g]], ssem0).wait()

        @pl.when(g + 2 < NCHUNK)
        def _():
            pltpu.async_copy(hn_hbm.at[sidx(g + 2)], buf0, gsem0)

        pltpu.make_async_copy(buf1, acc_sh.at[didx_v.at[g]], ssem1).wait()

        @pl.when(g + 3 < NCHUNK)
        def _():
            pltpu.async_copy(hn_hbm.at[sidx(g + 3)], buf1, gsem1)

        return 0

    lax.fori_loop(0, NCHUNK // 2, lambda i, x: pair(i * 2, x), 0)
    pltpu.make_async_copy(hn_hbm.at[sidx(NCHUNK - 1)], buf0, gsem0).wait()
    pltpu.async_copy(buf0, acc_sh.at[didx_v.at[NCHUNK - 1]], ssem0, add=True)
    pltpu.make_async_copy(buf0, acc_sh.at[didx_v.at[NCHUNK - 1]], ssem0).wait()
    plsc.subcore_barrier()
    pltpu.sync_copy(acc_sh.at[pl.ds(s * RPT, RPT)],
                    out_hbm.at[c, pl.ds(s * RPT, RPT)])


_sc_degree = functools.partial(
    pl.kernel,
    out_type=jax.ShapeDtypeStruct((NW, NP), jnp.float32),
    mesh=_mesh,
    scratch_types=[
        pltpu.VMEM((EPW,), jnp.int32),
        pltpu.VMEM((NP,), jnp.float32),
    ],
    compiler_params=pltpu.CompilerParams(needs_layout_passes=False),
)(_sc_degree_body)

_sc_aggregate = functools.partial(
    pl.kernel,
    out_type=jax.ShapeDtypeStruct((NC, NP, D), jnp.float32),
    mesh=_mesh,
    scratch_types=[
        pltpu.VMEM((EPW,), jnp.int32),
        pltpu.VMEM((NCHUNK, CH), jnp.int32),
        pltpu.VMEM((CH, D), jnp.float32),
        pltpu.VMEM((CH, D), jnp.float32),
        pltpu.VMEM_SHARED((NP, D), jnp.float32),
        pltpu.SemaphoreType.DMA,
        pltpu.SemaphoreType.DMA,
        pltpu.SemaphoreType.DMA,
        pltpu.SemaphoreType.DMA,
    ],
)(_sc_aggregate_body)


_RB = 1024
_GRID = NP // _RB


def _tc_proj_in_body(h_ref, w_ref, b_ref, deg_ref, hn_ref, nb_ref):
    deg = jnp.sum(deg_ref[...], axis=0)
    norm = lax.rsqrt(jnp.maximum(deg, 1.0))
    h0 = jnp.dot(h_ref[...], w_ref[...],
                 preferred_element_type=jnp.float32) + b_ref[...]
    normb = jnp.broadcast_to(norm[:, None], (_RB, D))
    hn_ref[...] = h0 * normb
    nb_ref[...] = normb


def _tc_layer_body(parts_ref, nb_ref, w_ref, b_ref, hn_ref):
    agg = (parts_ref[0] + parts_ref[1]) * nb_ref[...]
    t = jnp.dot(agg, w_ref[...], preferred_element_type=jnp.float32) + b_ref[...]
    h = t * jax.nn.sigmoid(t)
    hn_ref[...] = h * nb_ref[...]


def _tc_last_body(parts_ref, nb_ref, w_ref, b_ref, wo_ref, bo_ref, out_ref):
    agg = (parts_ref[0] + parts_ref[1]) * nb_ref[...]
    t = jnp.dot(agg, w_ref[...], preferred_element_type=jnp.float32) + b_ref[...]
    h = t * jax.nn.sigmoid(t)
    out_ref[...] = jnp.dot(h, wo_ref[...],
                           preferred_element_type=jnp.float32) + bo_ref[...]


_spec_rows = pl.BlockSpec((_RB, D), lambda i: (i, 0))
_spec_parts = pl.BlockSpec((NC, _RB, D), lambda i: (0, i, 0))
_spec_w = pl.BlockSpec((D, D), lambda i: (0, 0))
_spec_b = pl.BlockSpec((1, D), lambda i: (0, 0))
_spec_deg = pl.BlockSpec((NW, _RB), lambda i: (0, i))

_proj_in_call = pl.pallas_call(
    _tc_proj_in_body,
    grid=(_GRID,),
    in_specs=[_spec_rows, _spec_w, _spec_b, _spec_deg],
    out_specs=[_spec_rows, _spec_rows],
    out_shape=[jax.ShapeDtypeStruct((NP, D), jnp.float32),
               jax.ShapeDtypeStruct((NP, D), jnp.float32)],
)

_layer_call = pl.pallas_call(
    _tc_layer_body,
    grid=(_GRID,),
    in_specs=[_spec_parts, _spec_rows, _spec_w, _spec_b],
    out_specs=_spec_rows,
    out_shape=jax.ShapeDtypeStruct((NP, D), jnp.float32),
)

_last_call = pl.pallas_call(
    _tc_last_body,
    grid=(_GRID,),
    in_specs=[_spec_parts, _spec_rows, _spec_w, _spec_b, _spec_w, _spec_b],
    out_specs=_spec_rows,
    out_shape=jax.ShapeDtypeStruct((NP, D), jnp.float32),
)


@jax.jit
def kernel(h, edge_index, W_in, b_in, W_layers, b_layers, W_out, b_out):
    src = edge_index[0].reshape(NW, EPW)
    dst = edge_index[1].reshape(NW, NCHUNK, CH)
    dst_flat = edge_index[1].reshape(NW, EPW)
    zeros_d = jnp.zeros((RPT, D), jnp.float32)
    hp = jnp.pad(h, ((0, NP - N), (0, 0)))

    deg_parts = _sc_degree(dst_flat)
    hn, normb = _proj_in_call(hp, W_in, b_in.reshape(1, D), deg_parts)
    for i in range(DEPTH - 1):
        parts = _sc_aggregate(hn, src, dst, zeros_d)
        hn = _layer_call(parts, normb, W_layers[i], b_layers[i].reshape(1, D))
    parts = _sc_aggregate(hn, src, dst, zeros_d)
    out = _last_call(parts, normb, W_layers[DEPTH - 1],
                     b_layers[DEPTH - 1].reshape(1, D),
                     W_out, b_out.reshape(1, D))
    return out[:N]


# speedup vs baseline: 8.5699x; 1.0033x over previous
import functools

import jax
import jax.numpy as jnp
from jax import lax
from jax.experimental import pallas as pl
from jax.experimental.pallas import tpu as pltpu
from jax.experimental.pallas import tpu_sc as plsc

N = 10000
NP = 10240
E = 320000
D = 128
DEPTH = 4

NC = 2
NS = 16
NW = NC * NS
EPW = E // NW
CH = 80
NCHUNK = EPW // CH
RPT = NP // NS

_mesh = plsc.VectorSubcoreMesh(core_axis_name="c", subcore_axis_name="s")


def _sc_degree_body(dst_hbm, deg_out, didx_v, cnt_v):
    c = lax.axis_index("c")
    s = lax.axis_index("s")
    wid = s * NC + c
    pltpu.sync_copy(dst_hbm.at[wid], didx_v)
    zeros16 = jnp.zeros((16,), jnp.float32)
    ones16 = jnp.ones((16,), jnp.float32)

    def zero(i, _):
        cnt_v[pl.ds(i * 16, 16)] = zeros16
        return 0

    lax.fori_loop(0, NP // 16, zero, 0)

    def hist(i, _):
        dvec = didx_v[pl.ds(i * 16, 16)]
        plsc.addupdate_scatter(cnt_v, [dvec], ones16)
        return 0

    lax.fori_loop(0, EPW // 16, hist, 0)
    pltpu.sync_copy(cnt_v, deg_out.at[wid])


def _sc_aggregate_body(hn_hbm, src_hbm, dst_hbm, zeros_hbm, out_hbm,
                  sidx_v, didx_v, buf0, buf1, acc_sh,
                  gsem0, gsem1, ssem0, ssem1):
    c = lax.axis_index("c")
    s = lax.axis_index("s")
    wid = s * NC + c
    pltpu.sync_copy(src_hbm.at[wid], sidx_v)

    def sidx(j):
        return sidx_v.at[pl.ds(j * CH, CH)]

    def didx(j):
        return didx_v.at[pl.ds(j * CH, CH)]

    pltpu.async_copy(hn_hbm.at[sidx(0)], buf0, gsem0)
    pltpu.async_copy(hn_hbm.at[sidx(1)], buf1, gsem1)
    pltpu.sync_copy(dst_hbm.at[wid], didx_v)
    pltpu.sync_copy(zeros_hbm, acc_sh.at[pl.ds(s * RPT, RPT)])
    plsc.subcore_barrier()

    def pair(g, _):
        pltpu.make_async_copy(hn_hbm.at[sidx(g)], buf0, gsem0).wait()
        pltpu.async_copy(buf0, acc_sh.at[didx(g)], ssem0, add=True)
        pltpu.make_async_copy(hn_hbm.at[sidx(g + 1)], buf1, gsem1).wait()
        pltpu.async_copy(buf1, acc_sh.at[didx(g + 1)], ssem1, add=True)
        pltpu.make_async_copy(buf0, acc_sh.at[didx(g)], ssem0).wait()

        @pl.when(g + 2 < NCHUNK)
        def _():
            pltpu.async_copy(hn_hbm.at[sidx(g + 2)], buf0, gsem0)

        pltpu.make_async_copy(buf1, acc_sh.at[didx(g)], ssem1).wait()

        @pl.when(g + 3 < NCHUNK)
        def _():
            pltpu.async_copy(hn_hbm.at[sidx(g + 3)], buf1, gsem1)

        return 0

    lax.fori_loop(0, NCHUNK // 2, lambda i, x: pair(i * 2, x), 0)
    if NCHUNK % 2:
        pltpu.make_async_copy(hn_hbm.at[sidx(NCHUNK - 1)], buf0, gsem0).wait()
        pltpu.async_copy(buf0, acc_sh.at[didx(NCHUNK - 1)], ssem0, add=True)
        pltpu.make_async_copy(buf0, acc_sh.at[didx(NCHUNK - 1)], ssem0).wait()
    plsc.subcore_barrier()
    pltpu.sync_copy(acc_sh.at[pl.ds(s * RPT, RPT)],
                    out_hbm.at[c, pl.ds(s * RPT, RPT)])


_sc_degree = functools.partial(
    pl.kernel,
    out_type=jax.ShapeDtypeStruct((NW, NP), jnp.float32),
    mesh=_mesh,
    scratch_types=[
        pltpu.VMEM((EPW,), jnp.int32),
        pltpu.VMEM((NP,), jnp.float32),
    ],
    compiler_params=pltpu.CompilerParams(needs_layout_passes=False),
)(_sc_degree_body)

_sc_aggregate = functools.partial(
    pl.kernel,
    out_type=jax.ShapeDtypeStruct((NC, NP, D), jnp.float32),
    mesh=_mesh,
    scratch_types=[
        pltpu.VMEM((EPW,), jnp.int32),
        pltpu.VMEM((EPW,), jnp.int32),
        pltpu.VMEM((CH, D), jnp.float32),
        pltpu.VMEM((CH, D), jnp.float32),
        pltpu.VMEM_SHARED((NP, D), jnp.float32),
        pltpu.SemaphoreType.DMA,
        pltpu.SemaphoreType.DMA,
        pltpu.SemaphoreType.DMA,
        pltpu.SemaphoreType.DMA,
    ],
)(_sc_aggregate_body)


_RB = 1024
_GRID = NP // _RB


def _tc_proj_in_body(h_ref, w_ref, b_ref, deg_ref, hn_ref, nb_ref):
    deg = jnp.sum(deg_ref[...], axis=0)
    norm = lax.rsqrt(jnp.maximum(deg, 1.0))
    h0 = jnp.dot(h_ref[...], w_ref[...],
                 preferred_element_type=jnp.float32) + b_ref[...]
    normb = jnp.broadcast_to(norm[:, None], (_RB, D))
    hn_ref[...] = h0 * normb
    nb_ref[...] = normb


def _tc_layer_body(parts_ref, nb_ref, w_ref, b_ref, hn_ref):
    agg = (parts_ref[0] + parts_ref[1]) * nb_ref[...]
    t = jnp.dot(agg, w_ref[...], preferred_element_type=jnp.float32) + b_ref[...]
    h = t * jax.nn.sigmoid(t)
    hn_ref[...] = h * nb_ref[...]


def _tc_last_body(parts_ref, nb_ref, w_ref, b_ref, wo_ref, bo_ref, out_ref):
    agg = (parts_ref[0] + parts_ref[1]) * nb_ref[...]
    t = jnp.dot(agg, w_ref[...], preferred_element_type=jnp.float32) + b_ref[...]
    h = t * jax.nn.sigmoid(t)
    out_ref[...] = jnp.dot(h, wo_ref[...],
                           preferred_element_type=jnp.float32) + bo_ref[...]


_spec_rows = pl.BlockSpec((_RB, D), lambda i: (i, 0))
_spec_parts = pl.BlockSpec((NC, _RB, D), lambda i: (0, i, 0))
_spec_w = pl.BlockSpec((D, D), lambda i: (0, 0))
_spec_b = pl.BlockSpec((1, D), lambda i: (0, 0))
_spec_deg = pl.BlockSpec((NW, _RB), lambda i: (0, i))

_proj_in_call = pl.pallas_call(
    _tc_proj_in_body,
    grid=(_GRID,),
    in_specs=[_spec_rows, _spec_w, _spec_b, _spec_deg],
    out_specs=[_spec_rows, _spec_rows],
    out_shape=[jax.ShapeDtypeStruct((NP, D), jnp.float32),
               jax.ShapeDtypeStruct((NP, D), jnp.float32)],
)

_layer_call = pl.pallas_call(
    _tc_layer_body,
    grid=(_GRID,),
    in_specs=[_spec_parts, _spec_rows, _spec_w, _spec_b],
    out_specs=_spec_rows,
    out_shape=jax.ShapeDtypeStruct((NP, D), jnp.float32),
)

_last_call = pl.pallas_call(
    _tc_last_body,
    grid=(_GRID,),
    in_specs=[_spec_parts, _spec_rows, _spec_w, _spec_b, _spec_w, _spec_b],
    out_specs=_spec_rows,
    out_shape=jax.ShapeDtypeStruct((NP, D), jnp.float32),
)


@jax.jit
def kernel(h, edge_index, W_in, b_in, W_layers, b_layers, W_out, b_out):
    src = edge_index[0].reshape(NW, EPW)
    dst = edge_index[1].reshape(NW, EPW)
    dst_flat = dst
    zeros_d = jnp.zeros((RPT, D), jnp.float32)
    hp = jnp.pad(h, ((0, NP - N), (0, 0)))

    deg_parts = _sc_degree(dst_flat)
    hn, normb = _proj_in_call(hp, W_in, b_in.reshape(1, D), deg_parts)
    for i in range(DEPTH - 1):
        parts = _sc_aggregate(hn, src, dst, zeros_d)
        hn = _layer_call(parts, normb, W_layers[i], b_layers[i].reshape(1, D))
    parts = _sc_aggregate(hn, src, dst, zeros_d)
    out = _last_call(parts, normb, W_layers[DEPTH - 1],
                     b_layers[DEPTH - 1].reshape(1, D),
                     W_out, b_out.reshape(1, D))
    return out[:N]


# speedup vs baseline: 8.6292x; 1.0069x over previous
import functools

import jax
import jax.numpy as jnp
from jax import lax
from jax.experimental import pallas as pl
from jax.experimental.pallas import tpu as pltpu
from jax.experimental.pallas import tpu_sc as plsc

N = 10000
NP = 10240
E = 320000
D = 128
DEPTH = 4

NC = 2
NS = 16
NW = NC * NS
EPW = E // NW
CH = 80
NCHUNK = EPW // CH
RPT = NP // NS

_mesh = plsc.VectorSubcoreMesh(core_axis_name="c", subcore_axis_name="s")


def _sc_degree_body(dst_hbm, deg_out, didx_v, cnt_v):
    c = lax.axis_index("c")
    s = lax.axis_index("s")
    wid = s * NC + c
    pltpu.sync_copy(dst_hbm.at[wid], didx_v)
    zeros16 = jnp.zeros((16,), jnp.float32)
    ones16 = jnp.ones((16,), jnp.float32)

    def zero(i, _):
        cnt_v[pl.ds(i * 16, 16)] = zeros16
        return 0

    lax.fori_loop(0, NP // 16, zero, 0)

    def hist(i, _):
        dvec = didx_v[pl.ds(i * 16, 16)]
        plsc.addupdate_scatter(cnt_v, [dvec], ones16)
        return 0

    lax.fori_loop(0, EPW // 16, hist, 0)
    pltpu.sync_copy(cnt_v, deg_out.at[wid])


def _sc_aggregate_body(hn_hbm, src_hbm, dst_hbm, zeros_hbm, out_hbm,
                  sidx_v, didx_v, buf0, buf1, acc_sh,
                  gsem0, gsem1, ssem0, ssem1):
    c = lax.axis_index("c")
    s = lax.axis_index("s")
    wid = s * NC + c
    pltpu.sync_copy(src_hbm.at[wid], sidx_v)

    def sidx(j):
        return sidx_v.at[pl.ds(j * CH, CH)]

    def didx(j):
        return didx_v.at[pl.ds(j * CH, CH)]

    pltpu.async_copy(hn_hbm.at[sidx(0)], buf0, gsem0)
    pltpu.async_copy(hn_hbm.at[sidx(1)], buf1, gsem1)
    pltpu.sync_copy(dst_hbm.at[wid], didx_v)
    pltpu.sync_copy(zeros_hbm, acc_sh.at[pl.ds(s * RPT, RPT)])
    plsc.subcore_barrier()

    def pair(g, _):
        pltpu.make_async_copy(hn_hbm.at[sidx(g)], buf0, gsem0).wait()
        pltpu.async_copy(buf0, acc_sh.at[didx(g)], ssem0, add=True)
        pltpu.make_async_copy(hn_hbm.at[sidx(g + 1)], buf1, gsem1).wait()
        pltpu.async_copy(buf1, acc_sh.at[didx(g + 1)], ssem1, add=True)
        pltpu.make_async_copy(buf0, acc_sh.at[didx(g)], ssem0).wait()

        @pl.when(g + 2 < NCHUNK)
        def _():
            pltpu.async_copy(hn_hbm.at[sidx(g + 2)], buf0, gsem0)

        pltpu.make_async_copy(buf1, acc_sh.at[didx(g)], ssem1).wait()

        @pl.when(g + 3 < NCHUNK)
        def _():
            pltpu.async_copy(hn_hbm.at[sidx(g + 3)], buf1, gsem1)

        return 0

    lax.fori_loop(0, NCHUNK // 2, lambda i, x: pair(i * 2, x), 0)
    if NCHUNK % 2:
        pltpu.make_async_copy(hn_hbm.at[sidx(NCHUNK - 1)], buf0, gsem0).wait()
        pltpu.async_copy(buf0, acc_sh.at[didx(NCHUNK - 1)], ssem0, add=True)
        pltpu.make_async_copy(buf0, acc_sh.at[didx(NCHUNK - 1)], ssem0).wait()
    plsc.subcore_barrier()
    pltpu.sync_copy(acc_sh.at[pl.ds(s * RPT, RPT)],
                    out_hbm.at[c, pl.ds(s * RPT, RPT)])


_sc_degree = functools.partial(
    pl.kernel,
    out_type=jax.ShapeDtypeStruct((NW, NP), jnp.float32),
    mesh=_mesh,
    scratch_types=[
        pltpu.VMEM((EPW,), jnp.int32),
        pltpu.VMEM((NP,), jnp.float32),
    ],
    compiler_params=pltpu.CompilerParams(needs_layout_passes=False),
)(_sc_degree_body)

_sc_aggregate = functools.partial(
    pl.kernel,
    out_type=jax.ShapeDtypeStruct((NC, NP, D), jnp.float32),
    mesh=_mesh,
    scratch_types=[
        pltpu.VMEM((EPW,), jnp.int32),
        pltpu.VMEM((EPW,), jnp.int32),
        pltpu.VMEM((CH, D), jnp.float32),
        pltpu.VMEM((CH, D), jnp.float32),
        pltpu.VMEM_SHARED((NP, D), jnp.float32),
        pltpu.SemaphoreType.DMA,
        pltpu.SemaphoreType.DMA,
        pltpu.SemaphoreType.DMA,
        pltpu.SemaphoreType.DMA,
    ],
)(_sc_aggregate_body)


_RB = 1024
_GRID = NP // _RB


def _tc_proj_in_body(h_ref, w_ref, b_ref, deg_ref, hn_ref, nb_ref):
    deg = jnp.sum(deg_ref[...], axis=0)
    norm = lax.rsqrt(jnp.maximum(deg, 1.0))
    h0 = jnp.dot(h_ref[...], w_ref[...],
                 preferred_element_type=jnp.float32) + b_ref[...]
    normb = jnp.broadcast_to(norm[:, None], (_RB, D))
    hn_ref[...] = h0 * normb
    nb_ref[...] = normb


def _tc_layer_body(parts_ref, nb_ref, w_ref, b_ref, hn_ref):
    agg = (parts_ref[0] + parts_ref[1]) * nb_ref[...]
    t = jnp.dot(agg, w_ref[...], preferred_element_type=jnp.float32) + b_ref[...]
    h = t * jax.nn.sigmoid(t)
    hn_ref[...] = h * nb_ref[...]


def _tc_last_body(parts_ref, nb_ref, w_ref, b_ref, wo_ref, bo_ref, out_ref):
    agg = (parts_ref[0] + parts_ref[1]) * nb_ref[...]
    t = jnp.dot(agg, w_ref[...], preferred_element_type=jnp.float32) + b_ref[...]
    h = t * jax.nn.sigmoid(t)
    out_ref[...] = jnp.dot(h, wo_ref[...],
                           preferred_element_type=jnp.float32) + bo_ref[...]


_spec_rows = pl.BlockSpec((_RB, D), lambda i: (i, 0))
_spec_parts = pl.BlockSpec((NC, _RB, D), lambda i: (0, i, 0))
_spec_w = pl.BlockSpec((D, D), lambda i: (0, 0))
_spec_b = pl.BlockSpec((1, D), lambda i: (0, 0))
_spec_deg = pl.BlockSpec((NW, _RB), lambda i: (0, i))

_proj_in_call = pl.pallas_call(
    _tc_proj_in_body,
    grid=(_GRID,),
    in_specs=[_spec_rows, _spec_w, _spec_b, _spec_deg],
    out_specs=[_spec_rows, _spec_rows],
    out_shape=[jax.ShapeDtypeStruct((NP, D), jnp.float32),
               jax.ShapeDtypeStruct((NP, D), jnp.float32)],
)

_layer_call = pl.pallas_call(
    _tc_layer_body,
    grid=(_GRID,),
    in_specs=[_spec_parts, _spec_rows, _spec_w, _spec_b],
    out_specs=_spec_rows,
    out_shape=jax.ShapeDtypeStruct((NP, D), jnp.float32),
)

_last_call = pl.pallas_call(
    _tc_last_body,
    grid=(_GRID,),
    in_specs=[_spec_parts, _spec_rows, _spec_w, _spec_b, _spec_w, _spec_b],
    out_specs=_spec_rows,
    out_shape=jax.ShapeDtypeStruct((N, D), jnp.float32),
)


@jax.jit
def kernel(h, edge_index, W_in, b_in, W_layers, b_layers, W_out, b_out):
    src = edge_index[0].reshape(NW, EPW)
    dst = edge_index[1].reshape(NW, EPW)
    zeros_d = jnp.zeros((RPT, D), jnp.float32)

    deg_parts = _sc_degree(dst)
    hn, normb = _proj_in_call(h, W_in, b_in.reshape(1, D), deg_parts)
    for i in range(DEPTH - 1):
        parts = _sc_aggregate(hn, src, dst, zeros_d)
        hn = _layer_call(parts, normb, W_layers[i], b_layers[i].reshape(1, D))
    parts = _sc_aggregate(hn, src, dst, zeros_d)
    return _last_call(parts, normb, W_layers[DEPTH - 1],
                      b_layers[DEPTH - 1].reshape(1, D),
                      W_out, b_out.reshape(1, D))


# speedup vs baseline: 11.1034x; 1.2867x over previous
import functools

import jax
import jax.numpy as jnp
from jax import lax
from jax.experimental import pallas as pl
from jax.experimental.pallas import tpu as pltpu
from jax.experimental.pallas import tpu_sc as plsc

N = 10000
NP = 10240
E = 320000
D = 128
DEPTH = 4

NC = 2
NS = 16
NW = NC * NS
EPW = E // NW
CH = 40
NB = 5
NCHUNK = EPW // CH
RPT = NP // NS

_mesh = plsc.VectorSubcoreMesh(core_axis_name="c", subcore_axis_name="s")


def _sc_degree_body(dst_hbm, deg_out, didx_v, cnt_v):
    c = lax.axis_index("c")
    s = lax.axis_index("s")
    wid = s * NC + c
    pltpu.sync_copy(dst_hbm.at[wid], didx_v)
    zeros16 = jnp.zeros((16,), jnp.float32)
    ones16 = jnp.ones((16,), jnp.float32)

    def zero(i, _):
        cnt_v[pl.ds(i * 16, 16)] = zeros16
        return 0

    lax.fori_loop(0, NP // 16, zero, 0)

    def hist(i, _):
        dvec = didx_v[pl.ds(i * 16, 16)]
        plsc.addupdate_scatter(cnt_v, [dvec], ones16)
        return 0

    lax.fori_loop(0, EPW // 16, hist, 0)
    pltpu.sync_copy(cnt_v, deg_out.at[wid])


def _sc_aggregate_body(hn_hbm, src_hbm, dst_hbm, zeros_hbm, out_hbm,
                  sidx_v, didx_v, b0, b1, b2, b3, b4, acc_sh,
                  g0, g1, g2, g3, g4, s0, s1, s2, s3, s4):
    bufs = (b0, b1, b2, b3, b4)
    gsems = (g0, g1, g2, g3, g4)
    ssems = (s0, s1, s2, s3, s4)
    c = lax.axis_index("c")
    s = lax.axis_index("s")
    wid = s * NC + c
    pltpu.sync_copy(src_hbm.at[wid], sidx_v)

    def sidx(j):
        return sidx_v.at[pl.ds(j * CH, CH)]

    def didx(j):
        return didx_v.at[pl.ds(j * CH, CH)]

    for k in range(NB):
        pltpu.async_copy(hn_hbm.at[sidx(k)], bufs[k], gsems[k])
    pltpu.sync_copy(dst_hbm.at[wid], didx_v)
    pltpu.sync_copy(zeros_hbm, acc_sh.at[pl.ds(s * RPT, RPT)])
    plsc.subcore_barrier()

    def block(i, _):
        c0 = i * NB
        for k in range(NB):
            pltpu.make_async_copy(hn_hbm.at[sidx(c0 + k)], bufs[k],
                                  gsems[k]).wait()
            pltpu.async_copy(bufs[k], acc_sh.at[didx(c0 + k)], ssems[k],
                             add=True)
        for k in range(NB):
            pltpu.make_async_copy(bufs[k], acc_sh.at[didx(c0 + k)],
                                  ssems[k]).wait()

            @pl.when(c0 + NB + k < NCHUNK)
            def _():
                pltpu.async_copy(hn_hbm.at[sidx(c0 + NB + k)], bufs[k],
                                 gsems[k])

        return 0

    lax.fori_loop(0, NCHUNK // NB, block, 0)
    plsc.subcore_barrier()
    pltpu.sync_copy(acc_sh.at[pl.ds(s * RPT, RPT)],
                    out_hbm.at[c, pl.ds(s * RPT, RPT)])


_sc_degree = functools.partial(
    pl.kernel,
    out_type=jax.ShapeDtypeStruct((NW, NP), jnp.float32),
    mesh=_mesh,
    scratch_types=[
        pltpu.VMEM((EPW,), jnp.int32),
        pltpu.VMEM((NP,), jnp.float32),
    ],
    compiler_params=pltpu.CompilerParams(needs_layout_passes=False),
)(_sc_degree_body)

_sc_aggregate = functools.partial(
    pl.kernel,
    out_type=jax.ShapeDtypeStruct((NC, NP, D), jnp.float32),
    mesh=_mesh,
    scratch_types=(
        [pltpu.VMEM((EPW,), jnp.int32),
         pltpu.VMEM((EPW,), jnp.int32)]
        + [pltpu.VMEM((CH, D), jnp.float32)] * NB
        + [pltpu.VMEM_SHARED((NP, D), jnp.float32)]
        + [pltpu.SemaphoreType.DMA] * (2 * NB)
    ),
)(_sc_aggregate_body)


_RB = 1024
_GRID = NP // _RB


def _tc_proj_in_body(h_ref, w_ref, b_ref, deg_ref, hn_ref, nb_ref):
    deg = jnp.sum(deg_ref[...], axis=0)
    norm = lax.rsqrt(jnp.maximum(deg, 1.0))
    h0 = jnp.dot(h_ref[...], w_ref[...],
                 preferred_element_type=jnp.float32) + b_ref[...]
    normb = jnp.broadcast_to(norm[:, None], (_RB, D))
    hn_ref[...] = h0 * normb
    nb_ref[...] = normb


def _tc_layer_body(parts_ref, nb_ref, w_ref, b_ref, hn_ref):
    agg = (parts_ref[0] + parts_ref[1]) * nb_ref[...]
    t = jnp.dot(agg, w_ref[...], preferred_element_type=jnp.float32) + b_ref[...]
    h = t * jax.nn.sigmoid(t)
    hn_ref[...] = h * nb_ref[...]


def _tc_last_body(parts_ref, nb_ref, w_ref, b_ref, wo_ref, bo_ref, out_ref):
    agg = (parts_ref[0] + parts_ref[1]) * nb_ref[...]
    t = jnp.dot(agg, w_ref[...], preferred_element_type=jnp.float32) + b_ref[...]
    h = t * jax.nn.sigmoid(t)
    out_ref[...] = jnp.dot(h, wo_ref[...],
                           preferred_element_type=jnp.float32) + bo_ref[...]


_spec_rows = pl.BlockSpec((_RB, D), lambda i: (i, 0))
_spec_parts = pl.BlockSpec((NC, _RB, D), lambda i: (0, i, 0))
_spec_w = pl.BlockSpec((D, D), lambda i: (0, 0))
_spec_b = pl.BlockSpec((1, D), lambda i: (0, 0))
_spec_deg = pl.BlockSpec((NW, _RB), lambda i: (0, i))

_proj_in_call = pl.pallas_call(
    _tc_proj_in_body,
    grid=(_GRID,),
    in_specs=[_spec_rows, _spec_w, _spec_b, _spec_deg],
    out_specs=[_spec_rows, _spec_rows],
    out_shape=[jax.ShapeDtypeStruct((NP, D), jnp.float32),
               jax.ShapeDtypeStruct((NP, D), jnp.float32)],
)

_layer_call = pl.pallas_call(
    _tc_layer_body,
    grid=(_GRID,),
    in_specs=[_spec_parts, _spec_rows, _spec_w, _spec_b],
    out_specs=_spec_rows,
    out_shape=jax.ShapeDtypeStruct((NP, D), jnp.float32),
)

_last_call = pl.pallas_call(
    _tc_last_body,
    grid=(_GRID,),
    in_specs=[_spec_parts, _spec_rows, _spec_w, _spec_b, _spec_w, _spec_b],
    out_specs=_spec_rows,
    out_shape=jax.ShapeDtypeStruct((N, D), jnp.float32),
)


@jax.jit
def kernel(h, edge_index, W_in, b_in, W_layers, b_layers, W_out, b_out):
    src = edge_index[0].reshape(NW, EPW)
    dst = edge_index[1].reshape(NW, EPW)
    zeros_d = jnp.zeros((RPT, D), jnp.float32)

    deg_parts = _sc_degree(dst)
    hn, normb = _proj_in_call(h, W_in, b_in.reshape(1, D), deg_parts)
    for i in range(DEPTH - 1):
        parts = _sc_aggregate(hn, src, dst, zeros_d)
        hn = _layer_call(parts, normb, W_layers[i], b_layers[i].reshape(1, D))
    parts = _sc_aggregate(hn, src, dst, zeros_d)
    return _last_call(parts, normb, W_layers[DEPTH - 1],
                      b_layers[DEPTH - 1].reshape(1, D),
                      W_out, b_out.reshape(1, D))
